# Initial kernel scaffold; baseline (speedup 1.0000x reference)
import jax
import jax.numpy as jnp
from jax.experimental import pallas as pl


def kernel(idx_texts, table):
    raise NotImplementedError("write your pallas kernel here")


# speedup vs baseline: 1.4592x; 1.4592x over previous
import functools

import jax
import jax.numpy as jnp
from jax import lax
from jax.experimental import pallas as pl
from jax.experimental.pallas import tpu as pltpu
from jax.experimental.pallas import tpu_sc as plsc

EMBEDDING_DIM = 32
NUM_CORES = 2
NUM_SUBCORES = 16
NUM_WORKERS = NUM_CORES * NUM_SUBCORES
CHUNK = 1024


def _sc_gather(table, idx_flat, n_total):
    b_per_w = n_total // NUM_WORKERS
    n_chunks = b_per_w // CHUNK
    mesh = plsc.VectorSubcoreMesh(core_axis_name="c", subcore_axis_name="s")

    @functools.partial(
        pl.kernel,
        mesh=mesh,
        out_type=jax.ShapeDtypeStruct((n_total, EMBEDDING_DIM), jnp.float32),
        scratch_types=[
            pltpu.VMEM((CHUNK,), jnp.int32),
            pltpu.VMEM((CHUNK, EMBEDDING_DIM), jnp.float32),
            pltpu.SemaphoreType.DMA,
        ],
        compiler_params=pltpu.CompilerParams(use_tc_tiling_on_sc=False),
    )
    def k(table_hbm, idx_hbm, out_hbm, idx_v, rows_v, sem):
        wid = lax.axis_index("s") * NUM_CORES + lax.axis_index("c")
        base = wid * b_per_w

        def body(i, carry):
            off = base + i * CHUNK
            pltpu.sync_copy(idx_hbm.at[pl.ds(off, CHUNK)], idx_v)
            pltpu.async_copy(table_hbm.at[idx_v], rows_v, sem).wait()
            pltpu.sync_copy(rows_v, out_hbm.at[pl.ds(off, CHUNK)])
            return carry

        lax.fori_loop(0, n_chunks, body, 0)

    return k(table, idx_flat)


def kernel(idx_texts, table):
    n_total = idx_texts.shape[0] * idx_texts.shape[1]
    idx_flat = idx_texts.reshape(-1).astype(jnp.int32)
    out = _sc_gather(table, idx_flat, n_total)
    return out.reshape(idx_texts.shape[0], idx_texts.shape[1], EMBEDDING_DIM)


# speedup vs baseline: 1.5004x; 1.0282x over previous
import functools

import jax
import jax.numpy as jnp
from jax import lax
from jax.experimental import pallas as pl
from jax.experimental.pallas import tpu as pltpu
from jax.experimental.pallas import tpu_sc as plsc

EMBEDDING_DIM = 32
NUM_CORES = 2
NUM_SUBCORES = 16
NUM_WORKERS = NUM_CORES * NUM_SUBCORES
CHUNK = 1280
N_BUF = 2


def _sc_gather(table, idx_flat, n_total):
    b_per_w = n_total // NUM_WORKERS
    n_chunks = b_per_w // CHUNK
    n_outer = n_chunks // N_BUF
    mesh = plsc.VectorSubcoreMesh(core_axis_name="c", subcore_axis_name="s")

    @functools.partial(
        pl.kernel,
        mesh=mesh,
        out_type=jax.ShapeDtypeStruct((n_total, EMBEDDING_DIM), jnp.float32),
        scratch_types=[
            pltpu.VMEM((b_per_w,), jnp.int32),
            [pltpu.VMEM((CHUNK, EMBEDDING_DIM), jnp.float32) for _ in range(N_BUF)],
            [pltpu.SemaphoreType.DMA for _ in range(N_BUF)],
            [pltpu.SemaphoreType.DMA for _ in range(N_BUF)],
        ],
        compiler_params=pltpu.CompilerParams(use_tc_tiling_on_sc=False),
    )
    def k(table_hbm, idx_hbm, out_hbm, idx_v, rows, gsems, wsems):
        wid = lax.axis_index("s") * NUM_CORES + lax.axis_index("c")
        base = wid * b_per_w
        pltpu.sync_copy(idx_hbm.at[pl.ds(base, b_per_w)], idx_v)

        def g_start(c, p):
            pltpu.async_copy(
                table_hbm.at[idx_v.at[pl.ds(c * CHUNK, CHUNK)]], rows[p], gsems[p]
            )

        def g_wait(p):
            pltpu.make_async_copy(
                table_hbm.at[idx_v.at[pl.ds(0, CHUNK)]], rows[p], gsems[p]
            ).wait()

        def w_start(c, p):
            pltpu.async_copy(
                rows[p], out_hbm.at[pl.ds(base + c * CHUNK, CHUNK)], wsems[p]
            )

        def w_wait(p):
            pltpu.make_async_copy(
                rows[p], out_hbm.at[pl.ds(base, CHUNK)], wsems[p]
            ).wait()

        for p in range(N_BUF):
            g_start(p, p)

        def body(j, carry):
            for p in range(N_BUF):
                c = j * N_BUF + p
                g_wait(p)
                w_start(c, p)
                w_wait(p)
                g_start(c + N_BUF, p)
            return carry

        lax.fori_loop(0, n_outer - 1, body, 0)

        for p in range(N_BUF):
            g_wait(p)
            w_start((n_outer - 1) * N_BUF + p, p)
        for p in range(N_BUF):
            w_wait(p)

    return k(table, idx_flat)


def kernel(idx_texts, table):
    n_total = idx_texts.shape[0] * idx_texts.shape[1]
    idx_flat = idx_texts.reshape(-1).astype(jnp.int32)
    out = _sc_gather(table, idx_flat, n_total)
    return out.reshape(idx_texts.shape[0], idx_texts.shape[1], EMBEDDING_DIM)
